# baseline (speedup 1.0000x reference)
import functools

import jax
import jax.numpy as jnp
from jax import lax
from jax.experimental import pallas as pl
from jax.experimental.pallas import tpu as pltpu
from jax.experimental.pallas import tpu_sc as plsc

_NUM_CORES = 2
_NUM_SUBCORES = 16
_LANES = 16
_NW = _NUM_CORES * _NUM_SUBCORES
_ROW = 128


@functools.partial(jax.jit, static_argnames=("n",))
def _position_weights(offsets, position_weight, n):
    chunk = n // _NW
    vecs = chunk // _LANES
    rows = chunk // _ROW
    vecs_per_row = _ROW // _LANES
    mesh = plsc.VectorSubcoreMesh(core_axis_name="c", subcore_axis_name="s")

    @functools.partial(
        pl.kernel,
        mesh=mesh,
        out_type=jax.ShapeDtypeStruct((_NW * rows, _ROW), jnp.float32),
        compiler_params=pltpu.CompilerParams(needs_layout_passes=False),
        scratch_types=[
            pltpu.VMEM((_LANES,), jnp.int32),
            pltpu.VMEM((chunk,), jnp.int32),
            pltpu.VMEM((rows, _ROW), jnp.int32),
            pltpu.VMEM((rows, _ROW), jnp.float32),
            pltpu.SemaphoreType.DMA,
        ],
    )
    def body(offs_hbm, pw_hbm, out_hbm, offs_v, off_arr, idx_v, w_v, sem):
        wid = lax.axis_index("s") * _NUM_CORES + lax.axis_index("c")
        base = wid * chunk
        pltpu.sync_copy(offs_hbm.at[pl.ds(0, _LANES)], offs_v)

        offs_vec = offs_v[...]
        zero = jnp.zeros((_LANES,), jnp.int32)

        def zero_step(v, carry):
            off_arr[pl.ds(v * _LANES, _LANES)] = zero
            return carry

        lax.fori_loop(0, vecs, zero_step, 0)
        carry0 = jnp.max(jnp.where(offs_vec <= base, offs_vec, 0))
        in_tile = (offs_vec > base) & (offs_vec < base + chunk)
        plsc.store_scatter(off_arr, [offs_vec - base], offs_vec, mask=in_tile)

        lane = lax.iota(jnp.int32, _LANES)

        def step(v, carry):
            start = v * _LANES
            off = jnp.maximum(plsc.cummax(off_arr[pl.ds(start, _LANES)]), carry)
            seq = lane + (base + start) - off
            idx_v[v // vecs_per_row, pl.ds((v % vecs_per_row) * _LANES, _LANES)] = seq
            return off[_LANES - 1]

        lax.fori_loop(0, vecs, step, carry0)

        gathers = [
            pltpu.async_copy(pw_hbm.at[idx_v.at[r]], w_v.at[r], sem)
            for r in range(rows)
        ]
        for g in gathers:
            g.wait()
        pltpu.sync_copy(w_v, out_hbm.at[pl.ds(wid * rows, rows)])

    return body(offsets, position_weight)


def kernel(values, offsets, position_weight):
    n = values.shape[0]
    weights = _position_weights(offsets, position_weight, n).reshape(n)
    return values, offsets, weights


# speedup vs baseline: 1.2421x; 1.2421x over previous
import functools

import jax
import jax.numpy as jnp
from jax import lax
from jax.experimental import pallas as pl
from jax.experimental.pallas import tpu as pltpu
from jax.experimental.pallas import tpu_sc as plsc

_NUM_CORES = 2
_NUM_SUBCORES = 16
_LANES = 16
_NW = _NUM_CORES * _NUM_SUBCORES
_ROW = 128


@functools.partial(jax.jit, static_argnames=("n",))
def _position_weights(offsets, position_weight, n):
    chunk = n // _NW
    vecs = chunk // _LANES
    rows = chunk // _ROW
    vecs_per_row = _ROW // _LANES
    mesh = plsc.VectorSubcoreMesh(core_axis_name="c", subcore_axis_name="s")

    @functools.partial(
        pl.kernel,
        mesh=mesh,
        out_type=jax.ShapeDtypeStruct((n,), jnp.float32),
        compiler_params=pltpu.CompilerParams(needs_layout_passes=False),
        scratch_types=[
            pltpu.VMEM((_LANES,), jnp.int32),
            pltpu.VMEM((n,), jnp.float32),
            pltpu.VMEM((chunk,), jnp.int32),
            pltpu.VMEM((chunk,), jnp.float32),
            pltpu.SemaphoreType.DMA,
            pltpu.SemaphoreType.DMA,
        ],
    )
    def body(offs_hbm, pw_hbm, out_hbm, offs_v, pw_v, seq_arr, out_v, tsem, osem):
        wid = lax.axis_index("s") * _NUM_CORES + lax.axis_index("c")
        base = wid * chunk
        table_dma = pltpu.async_copy(pw_hbm, pw_v, tsem)
        pltpu.sync_copy(offs_hbm.at[pl.ds(0, _LANES)], offs_v)

        offs_vec = offs_v[...]
        zero = jnp.zeros((_LANES,), jnp.int32)

        def zero_step(v, carry):
            seq_arr[pl.ds(v * _LANES, _LANES)] = zero
            return carry

        lax.fori_loop(0, vecs, zero_step, 0)
        carry0 = jnp.max(jnp.where(offs_vec <= base, offs_vec, 0))
        in_tile = (offs_vec > base) & (offs_vec < base + chunk)
        plsc.store_scatter(seq_arr, [offs_vec - base], offs_vec, mask=in_tile)

        lane = lax.iota(jnp.int32, _LANES)

        def seq_step(v, carry):
            start = v * _LANES
            off = jnp.maximum(plsc.cummax(seq_arr[pl.ds(start, _LANES)]), carry)
            seq_arr[pl.ds(start, _LANES)] = lane + (base + start) - off
            return off[_LANES - 1]

        lax.fori_loop(0, vecs, seq_step, carry0)

        table_dma.wait()
        out_dmas = []
        for r in range(rows):
            for v in range(vecs_per_row):
                start = r * _ROW + v * _LANES
                out_v[start:start + _LANES] = plsc.load_gather(
                    pw_v, [seq_arr[start:start + _LANES]]
                )
            out_dmas.append(
                pltpu.async_copy(
                    out_v.at[pl.ds(r * _ROW, _ROW)],
                    out_hbm.at[pl.ds(base + r * _ROW, _ROW)],
                    osem,
                )
            )
        for d in out_dmas:
            d.wait()

    return body(offsets, position_weight)


def kernel(values, offsets, position_weight):
    n = values.shape[0]
    weights = _position_weights(offsets, position_weight, n)
    return values, offsets, weights


# speedup vs baseline: 1.3561x; 1.0918x over previous
import functools

import jax
import jax.numpy as jnp
from jax import lax
from jax.experimental import pallas as pl
from jax.experimental.pallas import tpu as pltpu
from jax.experimental.pallas import tpu_sc as plsc

_NUM_CORES = 1
_NUM_SUBCORES = 16
_LANES = 16
_NW = _NUM_CORES * _NUM_SUBCORES
_ROW = 128


@functools.partial(jax.jit, static_argnames=("n",))
def _position_weights(offsets, position_weight, n):
    chunk = n // _NW
    vecs = chunk // _LANES
    rows = chunk // _ROW
    vecs_per_row = _ROW // _LANES
    mesh = plsc.VectorSubcoreMesh(
        core_axis_name="c", subcore_axis_name="s", num_cores=_NUM_CORES
    )

    @functools.partial(
        pl.kernel,
        mesh=mesh,
        out_type=jax.ShapeDtypeStruct((n,), jnp.float32),
        compiler_params=pltpu.CompilerParams(needs_layout_passes=False),
        scratch_types=[
            pltpu.VMEM((_LANES,), jnp.int32),
            pltpu.VMEM((n,), jnp.float32),
            pltpu.VMEM((chunk,), jnp.int32),
            pltpu.VMEM((chunk,), jnp.float32),
            pltpu.SemaphoreType.DMA,
            pltpu.SemaphoreType.DMA,
        ],
    )
    def body(offs_hbm, pw_hbm, out_hbm, offs_v, pw_v, seq_arr, out_v, tsem, osem):
        wid = lax.axis_index("s") * _NUM_CORES + lax.axis_index("c")
        base = wid * chunk
        table_dma = pltpu.async_copy(pw_hbm, pw_v, tsem)
        pltpu.sync_copy(offs_hbm.at[pl.ds(0, _LANES)], offs_v)

        offs_vec = offs_v[...]
        zero = jnp.zeros((_LANES,), jnp.int32)

        def zero_step(v, carry):
            seq_arr[pl.ds(v * _LANES, _LANES)] = zero
            return carry

        lax.fori_loop(0, vecs, zero_step, 0)
        carry0 = jnp.max(jnp.where(offs_vec <= base, offs_vec, 0))
        in_tile = (offs_vec > base) & (offs_vec < base + chunk)
        plsc.store_scatter(seq_arr, [offs_vec - base], offs_vec, mask=in_tile)

        lane = lax.iota(jnp.int32, _LANES)

        def seq_step(v, carry):
            start = v * _LANES
            off = jnp.maximum(plsc.cummax(seq_arr[pl.ds(start, _LANES)]), carry)
            seq_arr[pl.ds(start, _LANES)] = lane + (base + start) - off
            return off[_LANES - 1]

        lax.fori_loop(0, vecs, seq_step, carry0)

        table_dma.wait()
        out_dmas = []
        for r in range(rows):
            for v in range(vecs_per_row):
                start = r * _ROW + v * _LANES
                out_v[start:start + _LANES] = plsc.load_gather(
                    pw_v, [seq_arr[start:start + _LANES]]
                )
            out_dmas.append(
                pltpu.async_copy(
                    out_v.at[pl.ds(r * _ROW, _ROW)],
                    out_hbm.at[pl.ds(base + r * _ROW, _ROW)],
                    osem,
                )
            )
        for d in out_dmas:
            d.wait()

    return body(offsets, position_weight)


def kernel(values, offsets, position_weight):
    n = values.shape[0]
    weights = _position_weights(offsets, position_weight, n)
    return values, offsets, weights
